# Initial kernel scaffold; baseline (speedup 1.0000x reference)
import jax
import jax.numpy as jnp
from jax.experimental import pallas as pl


def kernel(entity_embed, user_embed, relation_table, item_cf_embed, W1, W2, kg_src, kg_dst, edge_type, item_idx, user_idx):
    raise NotImplementedError("write your pallas kernel here")


# speedup vs baseline: 2.8499x; 2.8499x over previous
import jax
import jax.numpy as jnp
from jax import lax
from jax.experimental import pallas as pl
from jax.experimental.pallas import tpu as pltpu
from jax.experimental.pallas import tpu_sc as plsc

EPS = 1e-5
MAX_NORM = 1.0 - 1e-3
D = 128
NC, NS = 2, 16
NW = NC * NS
N_ENT = 10000
N_ITM = 6000
N_USR = 4000

E1 = 327680
R1 = E1 // 128
K1 = R1 // NW
E2 = 401408
R2 = E2 // 128
K2 = R2 // NW
G2, CH2 = 7, 14
NSEG = 10112
TRASH = 10000
F = NSEG // NS

def _mk_mesh():
    return plsc.VectorSubcoreMesh(core_axis_name="c", subcore_axis_name="s",
                                  num_cores=NC, num_subcores=NS)


def _wid():
    return lax.axis_index("s") * NC + lax.axis_index("c")


def _kg_gather_body(ent, src_i, dst_i, src_o, dst_o,
                    idx_s, idx_d, buf_s, buf_d, sem_s, sem_d):
    w = _wid()
    pltpu.sync_copy(src_i.at[w], idx_s)
    pltpu.sync_copy(dst_i.at[w], idx_d)

    def body(j, carry):
        cp1 = pltpu.async_copy(ent.at[idx_s.at[j]], buf_s, sem_s)
        cp2 = pltpu.async_copy(ent.at[idx_d.at[j]], buf_d, sem_d)
        cp1.wait()
        cp2.wait()
        off = w * (K1 * 128) + j * 128
        pltpu.sync_copy(buf_s, src_o.at[pl.ds(off, 128)])
        pltpu.sync_copy(buf_d, dst_o.at[pl.ds(off, 128)])
        return carry

    lax.fori_loop(0, K1, body, 0)


def _kg_gather(ent, src_i, dst_i):
    fn = pl.kernel(
        _kg_gather_body,
        out_type=(jax.ShapeDtypeStruct((E1, D), jnp.float32),
                  jax.ShapeDtypeStruct((E1, D), jnp.float32)),
        mesh=_mk_mesh(),
        scratch_types=[
            pltpu.VMEM((K1, 128), jnp.int32),
            pltpu.VMEM((K1, 128), jnp.int32),
            pltpu.VMEM((128, D), jnp.float32),
            pltpu.VMEM((128, D), jnp.float32),
            pltpu.SemaphoreType.DMA,
            pltpu.SemaphoreType.DMA,
        ],
    )
    return fn(ent, src_i, dst_i)


def _kg_scatter_body(tan, seg_i, zer_s, out_s, acc, idx_v, buf):
    c = lax.axis_index("c")
    s = lax.axis_index("s")
    w = s * NC + c
    pltpu.sync_copy(zer_s.at[pl.ds(s * F, F)], acc.at[pl.ds(s * F, F)])
    pltpu.sync_copy(seg_i.at[w], idx_v)
    plsc.subcore_barrier()

    def body(j, carry):
        off = w * (K1 * 128) + j * 128
        pltpu.sync_copy(tan.at[pl.ds(off, 128)], buf)
        pltpu.sync_copy(buf, acc.at[idx_v.at[j]], add=True)
        return carry

    lax.fori_loop(0, K1, body, 0)
    plsc.subcore_barrier()
    pltpu.sync_copy(acc.at[pl.ds(s * F, F)], out_s.at[c].at[pl.ds(s * F, F)])


def _kg_scatter(tan, seg_i, zer_s):
    fn = pl.kernel(
        _kg_scatter_body,
        out_type=jax.ShapeDtypeStruct((NC, NSEG, D), jnp.float32),
        mesh=_mk_mesh(),
        scratch_types=[
            pltpu.VMEM_SHARED((NSEG, D), jnp.float32),
            pltpu.VMEM((K1, 128), jnp.int32),
            pltpu.VMEM((128, D), jnp.float32),
        ],
    )
    return fn(tan, seg_i, zer_s)


def _int_body(node, src_i, dst_i, zer_s, out_s,
              acc, idx_s, idx_d, buf, sem):
    c = lax.axis_index("c")
    s = lax.axis_index("s")
    w = s * NC + c
    pltpu.sync_copy(zer_s.at[pl.ds(s * F, F)], acc.at[pl.ds(s * F, F)])
    plsc.subcore_barrier()

    def outer(g, carry):
        pltpu.sync_copy(src_i.at[w, g], idx_s)
        pltpu.sync_copy(dst_i.at[w, g], idx_d)

        def body(j, carry2):
            pltpu.async_copy(node.at[idx_s.at[j]], buf, sem).wait()
            pltpu.sync_copy(buf, acc.at[idx_d.at[j]], add=True)
            return carry2

        lax.fori_loop(0, CH2, body, 0)
        return carry

    lax.fori_loop(0, G2, outer, 0)
    plsc.subcore_barrier()
    pltpu.sync_copy(acc.at[pl.ds(s * F, F)], out_s.at[c].at[pl.ds(s * F, F)])


def _int_agg(node, src_i, dst_i, zer_s):
    fn = pl.kernel(
        _int_body,
        out_type=jax.ShapeDtypeStruct((NC, NSEG, D), jnp.float32),
        mesh=_mk_mesh(),
        scratch_types=[
            pltpu.VMEM_SHARED((NSEG, D), jnp.float32),
            pltpu.VMEM((CH2, 128), jnp.int32),
            pltpu.VMEM((CH2, 128), jnp.int32),
            pltpu.VMEM((128, D), jnp.float32),
            pltpu.SemaphoreType.DMA,
        ],
    )
    return fn(node, src_i, dst_i, zer_s)


def _cnt_body(seg_i, dst_i, ones_h, zer_s, out_c1, out_c2,
              cnt, idx1, idx2, ones_v):
    c = lax.axis_index("c")
    s = lax.axis_index("s")
    w = s * NC + c
    pltpu.sync_copy(zer_s.at[pl.ds(s * F, F)], cnt.at[pl.ds(s * F, F)])
    pltpu.sync_copy(ones_h, ones_v)
    pltpu.sync_copy(seg_i.at[w], idx1)
    plsc.subcore_barrier()

    def body1(j, carry):
        pltpu.sync_copy(ones_v, cnt.at[idx1.at[j]], add=True)
        return carry

    lax.fori_loop(0, K1, body1, 0)
    plsc.subcore_barrier()
    pltpu.sync_copy(cnt.at[pl.ds(s * F, F)], out_c1.at[c].at[pl.ds(s * F, F)])
    plsc.subcore_barrier()
    pltpu.sync_copy(zer_s.at[pl.ds(s * F, F)], cnt.at[pl.ds(s * F, F)])
    plsc.subcore_barrier()

    def outer(g, carry):
        pltpu.sync_copy(dst_i.at[w, g], idx2)

        def body2(j, carry2):
            pltpu.sync_copy(ones_v, cnt.at[idx2.at[j]], add=True)
            return carry2

        lax.fori_loop(0, CH2, body2, 0)
        return carry

    lax.fori_loop(0, G2, outer, 0)
    plsc.subcore_barrier()
    pltpu.sync_copy(cnt.at[pl.ds(s * F, F)], out_c2.at[c].at[pl.ds(s * F, F)])


def _counts(seg_i, dst_i, ones_h, zer_s):
    fn = pl.kernel(
        _cnt_body,
        out_type=(jax.ShapeDtypeStruct((NC, NSEG, D), jnp.float32),
                  jax.ShapeDtypeStruct((NC, NSEG, D), jnp.float32)),
        mesh=_mk_mesh(),
        scratch_types=[
            pltpu.VMEM_SHARED((NSEG, D), jnp.float32),
            pltpu.VMEM((K1, 128), jnp.int32),
            pltpu.VMEM((CH2, 128), jnp.int32),
            pltpu.VMEM((128, D), jnp.float32),
        ],
    )
    return fn(seg_i, dst_i, ones_h, zer_s)


def _sq(x):
    return jnp.sum(x * x, axis=-1, keepdims=True)


def _proj(x):
    n = jnp.sqrt(_sq(x) + 1e-15)
    return x * jnp.where(n > MAX_NORM, MAX_NORM / n, 1.0)


def _madd(x, y):
    x2 = _sq(x)
    y2 = _sq(y)
    xy = jnp.sum(x * y, axis=-1, keepdims=True)
    num = (1.0 + 2.0 * xy + y2) * x + (1.0 - x2) * y
    den = 1.0 + 2.0 * xy + x2 * y2
    return num / jnp.maximum(den, 1e-15)


BE = 1024


def _edge_body(src_ref, dst_ref, et_ref, rt_ref, out_ref):
    u = src_ref[...]
    p = dst_ref[...]
    et = et_ref[...]
    onehot = jnp.where(
        et + 2 == lax.broadcasted_iota(jnp.int32, (BE, 16), 1), 1.0, 0.0)
    rel = jnp.dot(onehot, rt_ref[...], preferred_element_type=jnp.float32)

    n = jnp.maximum(jnp.sqrt(_sq(u) + 1e-15), EPS)
    base = _proj(jnp.tanh(n) * u / n)
    lam = 2.0 / jnp.maximum(1.0 - _sq(base), EPS)

    def emap(v):
        nv = jnp.maximum(jnp.sqrt(_sq(v) + 1e-15), EPS)
        sec = jnp.tanh(lam * nv / 2.0) * v / nv
        return _proj(_madd(base, sec))

    m = _proj(_madd(emap(p), emap(rel)))
    sub = _madd(-base, m)
    ns = jnp.clip(jnp.sqrt(_sq(sub) + 1e-15), EPS, 1.0 - 1e-5)
    atanh = 0.5 * jnp.log((1.0 + ns) / (1.0 - ns))
    out_ref[...] = (2.0 / lam) * atanh * sub / ns


def _edge_transform(src_p, dst_p, et2, reltab):
    grid = (E1 // BE,)
    return pl.pallas_call(
        _edge_body,
        grid=grid,
        in_specs=[
            pl.BlockSpec((BE, D), lambda i: (i, 0)),
            pl.BlockSpec((BE, D), lambda i: (i, 0)),
            pl.BlockSpec((BE, 1), lambda i: (i, 0)),
            pl.BlockSpec((16, D), lambda i: (0, 0)),
        ],
        out_specs=pl.BlockSpec((BE, D), lambda i: (i, 0)),
        out_shape=jax.ShapeDtypeStruct((E1, D), jnp.float32),
    )(src_p, dst_p, et2, reltab)


RF = 1000


def _fuse_body(e_ref, cf_ref, w1_ref, w2_ref, out_ref):
    e = e_ref[...]
    cf = cf_ref[...]
    g = jax.nn.sigmoid(
        jnp.dot(e, w1_ref[...], preferred_element_type=jnp.float32)
        + jnp.dot(cf, w2_ref[...], preferred_element_type=jnp.float32))
    out_ref[...] = g * e + (1.0 - g) * cf


def _fusion(ent_itm, cf, w1t, w2t):
    return pl.pallas_call(
        _fuse_body,
        grid=(N_ITM // RF,),
        in_specs=[
            pl.BlockSpec((RF, D), lambda i: (i, 0)),
            pl.BlockSpec((RF, D), lambda i: (i, 0)),
            pl.BlockSpec((D, D), lambda i: (0, 0)),
            pl.BlockSpec((D, D), lambda i: (0, 0)),
        ],
        out_specs=pl.BlockSpec((RF, D), lambda i: (i, 0)),
        out_shape=jax.ShapeDtypeStruct((N_ITM, D), jnp.float32),
    )(ent_itm, cf, w1t, w2t)


RB = 2528


def _final_body(s1_ref, c1_ref, s2_ref, c2_ref, o1_ref, o2_ref):
    s1 = s1_ref[0] + s1_ref[1]
    c1 = c1_ref[0, :, 0:1] + c1_ref[1, :, 0:1]
    o1_ref[...] = s1 / jnp.maximum(c1, 1.0)
    s2 = s2_ref[0] + s2_ref[1]
    c2 = c2_ref[0, :, 0:1] + c2_ref[1, :, 0:1]
    o2_ref[...] = s2 / jnp.maximum(c2, 1.0)


def _finalize(s1, c1, s2, c2):
    return pl.pallas_call(
        _final_body,
        grid=(NSEG // RB,),
        in_specs=[
            pl.BlockSpec((NC, RB, D), lambda i: (0, i, 0)),
            pl.BlockSpec((NC, RB, D), lambda i: (0, i, 0)),
            pl.BlockSpec((NC, RB, D), lambda i: (0, i, 0)),
            pl.BlockSpec((NC, RB, D), lambda i: (0, i, 0)),
        ],
        out_specs=(pl.BlockSpec((RB, D), lambda i: (i, 0)),
                   pl.BlockSpec((RB, D), lambda i: (i, 0))),
        out_shape=(jax.ShapeDtypeStruct((NSEG, D), jnp.float32),
                   jax.ShapeDtypeStruct((NSEG, D), jnp.float32)),
    )(s1, c1, s2, c2)


@jax.jit
def kernel(entity_embed, user_embed, relation_table, item_cf_embed, W1, W2,
           kg_src, kg_dst, edge_type, item_idx, user_idx):
    e_kg = kg_src.shape[0]
    e_int = item_idx.shape[0]

    pad1 = E1 - e_kg
    src_i = jnp.concatenate([kg_src, jnp.zeros((pad1,), jnp.int32)])
    dst_i = jnp.concatenate([kg_dst, jnp.zeros((pad1,), jnp.int32)])
    seg_i = jnp.concatenate([kg_src, jnp.full((pad1,), TRASH, jnp.int32)])
    et2 = jnp.concatenate([edge_type, jnp.zeros((pad1,), jnp.int32)])
    src_i = src_i.reshape(NW, K1, 128)
    dst_i = dst_i.reshape(NW, K1, 128)
    seg_i = seg_i.reshape(NW, K1, 128)
    et2 = et2.reshape(E1, 1)

    pad2 = E2 - 2 * e_int
    src2 = jnp.concatenate([item_idx, user_idx + N_ITM,
                            jnp.full((pad2,), TRASH, jnp.int32)])
    dst2 = jnp.concatenate([user_idx + N_ITM, item_idx,
                            jnp.full((pad2,), TRASH, jnp.int32)])
    src2 = src2.reshape(NW, G2, CH2, 128)
    dst2 = dst2.reshape(NW, G2, CH2, 128)

    reltab = jnp.concatenate(
        [relation_table, jnp.zeros((16 - relation_table.shape[0], D),
                                   jnp.float32)])
    ones_h = jnp.ones((128, D), jnp.float32)
    zer_s = jnp.zeros((NSEG, D), jnp.float32)

    src_p, dst_p = _kg_gather(entity_embed, src_i, dst_i)

    c1, c2 = _counts(seg_i, dst2, ones_h, zer_s)

    fus = _fusion(entity_embed[:N_ITM], item_cf_embed, W1.T, W2.T)
    node = jnp.concatenate([fus, user_embed,
                            jnp.zeros((NSEG - N_ITM - N_USR, D),
                                      jnp.float32)])

    tan = _edge_transform(src_p, dst_p, et2, reltab)

    s1 = _kg_scatter(tan, seg_i, zer_s)

    s2 = _int_agg(node, src2, dst2, zer_s)

    o1, o2 = _finalize(s1, c1, s2, c2)

    out = o1[:N_ENT]
    u = o2[N_ITM:N_ITM + N_USR]
    i_cf = o2[:N_ITM]
    return (out, u, i_cf)


# speedup vs baseline: 2.9584x; 1.0381x over previous
import jax
import jax.numpy as jnp
from jax import lax
from jax.experimental import pallas as pl
from jax.experimental.pallas import tpu as pltpu
from jax.experimental.pallas import tpu_sc as plsc

EPS = 1e-5
MAX_NORM = 1.0 - 1e-3
D = 128
NC, NS = 2, 16
NW = NC * NS
N_ENT = 10000
N_ITM = 6000
N_USR = 4000

E1 = 327680
R1 = E1 // 128
K1 = R1 // NW
E2 = 401408
R2 = E2 // 128
K2 = R2 // NW
G2, CH2 = 7, 14
NSEG = 10112
TRASH = 10000
F = NSEG // NS

def _mk_mesh():
    return plsc.VectorSubcoreMesh(core_axis_name="c", subcore_axis_name="s",
                                  num_cores=NC, num_subcores=NS)


def _wid():
    return lax.axis_index("s") * NC + lax.axis_index("c")


def _kg_gather_body(ent, src_i, dst_i, src_o, dst_o,
                    idx_s, idx_d, bs0, bs1, bd0, bd1, g0, g1, w0, w1):
    w = _wid()
    pltpu.sync_copy(src_i.at[w], idx_s)
    pltpu.sync_copy(dst_i.at[w], idx_d)
    base = w * (K1 * 128)
    H = K1 // 2

    pltpu.make_async_copy(ent.at[idx_s.at[0]], bs0, g0).start()
    pltpu.make_async_copy(ent.at[idx_d.at[0]], bd0, g0).start()

    def grp(g, carry):
        j0 = 2 * g
        j1 = j0 + 1
        off0 = base + j0 * 128
        off1 = off0 + 128

        @pl.when(g > 0)
        def _():
            pltpu.make_async_copy(bs1, src_o.at[pl.ds(off1, 128)], w1).wait()
            pltpu.make_async_copy(bd1, dst_o.at[pl.ds(off1, 128)], w1).wait()

        pltpu.make_async_copy(ent.at[idx_s.at[j1]], bs1, g1).start()
        pltpu.make_async_copy(ent.at[idx_d.at[j1]], bd1, g1).start()

        pltpu.make_async_copy(ent.at[idx_s.at[j0]], bs0, g0).wait()
        pltpu.make_async_copy(ent.at[idx_d.at[j0]], bd0, g0).wait()
        pltpu.make_async_copy(bs0, src_o.at[pl.ds(off0, 128)], w0).start()
        pltpu.make_async_copy(bd0, dst_o.at[pl.ds(off0, 128)], w0).start()

        @pl.when(g < H - 1)
        def _():
            pltpu.make_async_copy(bs0, src_o.at[pl.ds(off0, 128)], w0).wait()
            pltpu.make_async_copy(bd0, dst_o.at[pl.ds(off0, 128)], w0).wait()
            pltpu.make_async_copy(ent.at[idx_s.at[j0 + 2]], bs0, g0).start()
            pltpu.make_async_copy(ent.at[idx_d.at[j0 + 2]], bd0, g0).start()

        pltpu.make_async_copy(ent.at[idx_s.at[j1]], bs1, g1).wait()
        pltpu.make_async_copy(ent.at[idx_d.at[j1]], bd1, g1).wait()
        pltpu.make_async_copy(bs1, src_o.at[pl.ds(off1, 128)], w1).start()
        pltpu.make_async_copy(bd1, dst_o.at[pl.ds(off1, 128)], w1).start()
        return carry

    lax.fori_loop(0, H, grp, 0)

    endo = base + (K1 - 2) * 128
    pltpu.make_async_copy(bs0, src_o.at[pl.ds(endo, 128)], w0).wait()
    pltpu.make_async_copy(bd0, dst_o.at[pl.ds(endo, 128)], w0).wait()
    pltpu.make_async_copy(bs1, src_o.at[pl.ds(endo + 128, 128)], w1).wait()
    pltpu.make_async_copy(bd1, dst_o.at[pl.ds(endo + 128, 128)], w1).wait()


def _kg_gather(ent, src_i, dst_i):
    fn = pl.kernel(
        _kg_gather_body,
        out_type=(jax.ShapeDtypeStruct((E1, D), jnp.float32),
                  jax.ShapeDtypeStruct((E1, D), jnp.float32)),
        mesh=_mk_mesh(),
        name="sc_kg_gather",
        scratch_types=[
            pltpu.VMEM((K1, 128), jnp.int32),
            pltpu.VMEM((K1, 128), jnp.int32),
            pltpu.VMEM((128, D), jnp.float32),
            pltpu.VMEM((128, D), jnp.float32),
            pltpu.VMEM((128, D), jnp.float32),
            pltpu.VMEM((128, D), jnp.float32),
            pltpu.SemaphoreType.DMA,
            pltpu.SemaphoreType.DMA,
            pltpu.SemaphoreType.DMA,
            pltpu.SemaphoreType.DMA,
        ],
    )
    return fn(ent, src_i, dst_i)


def _kg_scatter_body(tan, seg_i, zer_s, out_s, acc, idx_v, buf):
    c = lax.axis_index("c")
    s = lax.axis_index("s")
    w = s * NC + c
    pltpu.sync_copy(zer_s.at[pl.ds(s * F, F)], acc.at[pl.ds(s * F, F)])
    pltpu.sync_copy(seg_i.at[w], idx_v)
    plsc.subcore_barrier()

    def body(j, carry):
        off = w * (K1 * 128) + j * 128
        pltpu.sync_copy(tan.at[pl.ds(off, 128)], buf)
        pltpu.sync_copy(buf, acc.at[idx_v.at[j]], add=True)
        return carry

    lax.fori_loop(0, K1, body, 0)
    plsc.subcore_barrier()
    pltpu.sync_copy(acc.at[pl.ds(s * F, F)], out_s.at[c].at[pl.ds(s * F, F)])


def _kg_scatter(tan, seg_i, zer_s):
    fn = pl.kernel(
        _kg_scatter_body,
        out_type=jax.ShapeDtypeStruct((NC, NSEG, D), jnp.float32),
        mesh=_mk_mesh(),
        name="sc_kg_scatter",
        scratch_types=[
            pltpu.VMEM_SHARED((NSEG, D), jnp.float32),
            pltpu.VMEM((K1, 128), jnp.int32),
            pltpu.VMEM((128, D), jnp.float32),
        ],
    )
    return fn(tan, seg_i, zer_s)


def _int_body(node, src_i, dst_i, zer_s, out_s,
              acc, idx_s, idx_d, buf, sem):
    c = lax.axis_index("c")
    s = lax.axis_index("s")
    w = s * NC + c
    pltpu.sync_copy(zer_s.at[pl.ds(s * F, F)], acc.at[pl.ds(s * F, F)])
    plsc.subcore_barrier()

    def outer(g, carry):
        pltpu.sync_copy(src_i.at[w, g], idx_s)
        pltpu.sync_copy(dst_i.at[w, g], idx_d)

        def body(j, carry2):
            pltpu.async_copy(node.at[idx_s.at[j]], buf, sem).wait()
            pltpu.sync_copy(buf, acc.at[idx_d.at[j]], add=True)
            return carry2

        lax.fori_loop(0, CH2, body, 0)
        return carry

    lax.fori_loop(0, G2, outer, 0)
    plsc.subcore_barrier()
    pltpu.sync_copy(acc.at[pl.ds(s * F, F)], out_s.at[c].at[pl.ds(s * F, F)])


def _int_agg(node, src_i, dst_i, zer_s):
    fn = pl.kernel(
        _int_body,
        out_type=jax.ShapeDtypeStruct((NC, NSEG, D), jnp.float32),
        mesh=_mk_mesh(),
        name="sc_int_agg",
        scratch_types=[
            pltpu.VMEM_SHARED((NSEG, D), jnp.float32),
            pltpu.VMEM((CH2, 128), jnp.int32),
            pltpu.VMEM((CH2, 128), jnp.int32),
            pltpu.VMEM((128, D), jnp.float32),
            pltpu.SemaphoreType.DMA,
        ],
    )
    return fn(node, src_i, dst_i, zer_s)


def _cnt_body(seg_i, dst_i, ones_h, zer_s, out_c1, out_c2,
              cnt, idx1, idx2, ones_v):
    c = lax.axis_index("c")
    s = lax.axis_index("s")
    w = s * NC + c
    pltpu.sync_copy(zer_s.at[pl.ds(s * F, F)], cnt.at[pl.ds(s * F, F)])
    pltpu.sync_copy(ones_h, ones_v)
    pltpu.sync_copy(seg_i.at[w], idx1)
    plsc.subcore_barrier()

    def body1(j, carry):
        pltpu.sync_copy(ones_v, cnt.at[idx1.at[j]], add=True)
        return carry

    lax.fori_loop(0, K1, body1, 0)
    plsc.subcore_barrier()
    pltpu.sync_copy(cnt.at[pl.ds(s * F, F)], out_c1.at[c].at[pl.ds(s * F, F)])
    plsc.subcore_barrier()
    pltpu.sync_copy(zer_s.at[pl.ds(s * F, F)], cnt.at[pl.ds(s * F, F)])
    plsc.subcore_barrier()

    def outer(g, carry):
        pltpu.sync_copy(dst_i.at[w, g], idx2)

        def body2(j, carry2):
            pltpu.sync_copy(ones_v, cnt.at[idx2.at[j]], add=True)
            return carry2

        lax.fori_loop(0, CH2, body2, 0)
        return carry

    lax.fori_loop(0, G2, outer, 0)
    plsc.subcore_barrier()
    pltpu.sync_copy(cnt.at[pl.ds(s * F, F)], out_c2.at[c].at[pl.ds(s * F, F)])


def _counts(seg_i, dst_i, ones_h, zer_s):
    fn = pl.kernel(
        _cnt_body,
        out_type=(jax.ShapeDtypeStruct((NC, NSEG, D), jnp.float32),
                  jax.ShapeDtypeStruct((NC, NSEG, D), jnp.float32)),
        mesh=_mk_mesh(),
        name="sc_counts",
        scratch_types=[
            pltpu.VMEM_SHARED((NSEG, D), jnp.float32),
            pltpu.VMEM((K1, 128), jnp.int32),
            pltpu.VMEM((CH2, 128), jnp.int32),
            pltpu.VMEM((128, D), jnp.float32),
        ],
    )
    return fn(seg_i, dst_i, ones_h, zer_s)


def _sq(x):
    return jnp.sum(x * x, axis=-1, keepdims=True)


def _proj(x):
    n = jnp.sqrt(_sq(x) + 1e-15)
    return x * jnp.where(n > MAX_NORM, MAX_NORM / n, 1.0)


def _madd(x, y):
    x2 = _sq(x)
    y2 = _sq(y)
    xy = jnp.sum(x * y, axis=-1, keepdims=True)
    num = (1.0 + 2.0 * xy + y2) * x + (1.0 - x2) * y
    den = 1.0 + 2.0 * xy + x2 * y2
    return num / jnp.maximum(den, 1e-15)


BE = 1024


def _edge_body(src_ref, dst_ref, et_ref, rt_ref, out_ref):
    u = src_ref[...]
    p = dst_ref[...]
    et = et_ref[...]
    onehot = jnp.where(
        et + 2 == lax.broadcasted_iota(jnp.int32, (BE, 16), 1), 1.0, 0.0)
    rel = jnp.dot(onehot, rt_ref[...], preferred_element_type=jnp.float32)

    n = jnp.maximum(jnp.sqrt(_sq(u) + 1e-15), EPS)
    base = _proj(jnp.tanh(n) * u / n)
    lam = 2.0 / jnp.maximum(1.0 - _sq(base), EPS)

    def emap(v):
        nv = jnp.maximum(jnp.sqrt(_sq(v) + 1e-15), EPS)
        sec = jnp.tanh(lam * nv / 2.0) * v / nv
        return _proj(_madd(base, sec))

    m = _proj(_madd(emap(p), emap(rel)))
    sub = _madd(-base, m)
    ns = jnp.clip(jnp.sqrt(_sq(sub) + 1e-15), EPS, 1.0 - 1e-5)
    atanh = 0.5 * jnp.log((1.0 + ns) / (1.0 - ns))
    out_ref[...] = (2.0 / lam) * atanh * sub / ns


def _edge_transform(src_p, dst_p, et2, reltab):
    grid = (E1 // BE,)
    return pl.pallas_call(
        _edge_body,
        grid=grid,
        in_specs=[
            pl.BlockSpec((BE, D), lambda i: (i, 0)),
            pl.BlockSpec((BE, D), lambda i: (i, 0)),
            pl.BlockSpec((BE, 1), lambda i: (i, 0)),
            pl.BlockSpec((16, D), lambda i: (0, 0)),
        ],
        out_specs=pl.BlockSpec((BE, D), lambda i: (i, 0)),
        out_shape=jax.ShapeDtypeStruct((E1, D), jnp.float32),
    )(src_p, dst_p, et2, reltab)


RF = 1000


def _fuse_body(e_ref, cf_ref, w1_ref, w2_ref, out_ref):
    e = e_ref[...]
    cf = cf_ref[...]
    g = jax.nn.sigmoid(
        jnp.dot(e, w1_ref[...], preferred_element_type=jnp.float32)
        + jnp.dot(cf, w2_ref[...], preferred_element_type=jnp.float32))
    out_ref[...] = g * e + (1.0 - g) * cf


def _fusion(ent_itm, cf, w1t, w2t):
    return pl.pallas_call(
        _fuse_body,
        grid=(N_ITM // RF,),
        in_specs=[
            pl.BlockSpec((RF, D), lambda i: (i, 0)),
            pl.BlockSpec((RF, D), lambda i: (i, 0)),
            pl.BlockSpec((D, D), lambda i: (0, 0)),
            pl.BlockSpec((D, D), lambda i: (0, 0)),
        ],
        out_specs=pl.BlockSpec((RF, D), lambda i: (i, 0)),
        out_shape=jax.ShapeDtypeStruct((N_ITM, D), jnp.float32),
    )(ent_itm, cf, w1t, w2t)


RB = 2528


def _final_body(s1_ref, c1_ref, s2_ref, c2_ref, o1_ref, o2_ref):
    s1 = s1_ref[0] + s1_ref[1]
    c1 = c1_ref[0, :, 0:1] + c1_ref[1, :, 0:1]
    o1_ref[...] = s1 / jnp.maximum(c1, 1.0)
    s2 = s2_ref[0] + s2_ref[1]
    c2 = c2_ref[0, :, 0:1] + c2_ref[1, :, 0:1]
    o2_ref[...] = s2 / jnp.maximum(c2, 1.0)


def _finalize(s1, c1, s2, c2):
    return pl.pallas_call(
        _final_body,
        grid=(NSEG // RB,),
        in_specs=[
            pl.BlockSpec((NC, RB, D), lambda i: (0, i, 0)),
            pl.BlockSpec((NC, RB, D), lambda i: (0, i, 0)),
            pl.BlockSpec((NC, RB, D), lambda i: (0, i, 0)),
            pl.BlockSpec((NC, RB, D), lambda i: (0, i, 0)),
        ],
        out_specs=(pl.BlockSpec((RB, D), lambda i: (i, 0)),
                   pl.BlockSpec((RB, D), lambda i: (i, 0))),
        out_shape=(jax.ShapeDtypeStruct((NSEG, D), jnp.float32),
                   jax.ShapeDtypeStruct((NSEG, D), jnp.float32)),
    )(s1, c1, s2, c2)


@jax.jit
def kernel(entity_embed, user_embed, relation_table, item_cf_embed, W1, W2,
           kg_src, kg_dst, edge_type, item_idx, user_idx):
    e_kg = kg_src.shape[0]
    e_int = item_idx.shape[0]

    pad1 = E1 - e_kg
    src_i = jnp.concatenate([kg_src, jnp.zeros((pad1,), jnp.int32)])
    dst_i = jnp.concatenate([kg_dst, jnp.zeros((pad1,), jnp.int32)])
    seg_i = jnp.concatenate([kg_src, jnp.full((pad1,), TRASH, jnp.int32)])
    et2 = jnp.concatenate([edge_type, jnp.zeros((pad1,), jnp.int32)])
    src_i = src_i.reshape(NW, K1, 128)
    dst_i = dst_i.reshape(NW, K1, 128)
    seg_i = seg_i.reshape(NW, K1, 128)
    et2 = et2.reshape(E1, 1)

    pad2 = E2 - 2 * e_int
    src2 = jnp.concatenate([item_idx, user_idx + N_ITM,
                            jnp.full((pad2,), TRASH, jnp.int32)])
    dst2 = jnp.concatenate([user_idx + N_ITM, item_idx,
                            jnp.full((pad2,), TRASH, jnp.int32)])
    src2 = src2.reshape(NW, G2, CH2, 128)
    dst2 = dst2.reshape(NW, G2, CH2, 128)

    reltab = jnp.concatenate(
        [relation_table, jnp.zeros((16 - relation_table.shape[0], D),
                                   jnp.float32)])
    ones_h = jnp.ones((128, D), jnp.float32)
    zer_s = jnp.zeros((NSEG, D), jnp.float32)

    src_p, dst_p = _kg_gather(entity_embed, src_i, dst_i)

    c1, c2 = _counts(seg_i, dst2, ones_h, zer_s)

    fus = _fusion(entity_embed[:N_ITM], item_cf_embed, W1.T, W2.T)
    node = jnp.concatenate([fus, user_embed,
                            jnp.zeros((NSEG - N_ITM - N_USR, D),
                                      jnp.float32)])

    tan = _edge_transform(src_p, dst_p, et2, reltab)

    s1 = _kg_scatter(tan, seg_i, zer_s)

    s2 = _int_agg(node, src2, dst2, zer_s)

    o1, o2 = _finalize(s1, c1, s2, c2)

    out = o1[:N_ENT]
    u = o2[N_ITM:N_ITM + N_USR]
    i_cf = o2[:N_ITM]
    return (out, u, i_cf)
